# baseline (speedup 1.0000x reference)
import functools

import jax
import jax.numpy as jnp
from jax.experimental import pallas as pl
from jax.experimental.pallas import tpu as pltpu

EPSV = 1e-12
HC = 16
HIGH = jax.lax.Precision.HIGHEST
NSPLIT = 4


def _slab_copies(hbm, b, f_bufs, slot, sems):
    C = f_bufs.shape[1]
    cs = C // NSPLIT
    return [
        pltpu.make_async_copy(hbm.at[b, pl.ds(s * cs, cs)],
                              f_bufs.at[slot, pl.ds(s * cs, cs)],
                              sems.at[slot])
        for s in range(NSPLIT)
    ]


def _out_copies(f_bufs, slot, hbm, b, sems):
    C = f_bufs.shape[1]
    cs = C // NSPLIT
    return [
        pltpu.make_async_copy(f_bufs.at[slot, pl.ds(s * cs, cs)],
                              hbm.at[b, pl.ds(s * cs, cs)],
                              sems.at[slot])
        for s in range(NSPLIT)
    ]


def _step(j, core, per, f_hbm, out_hbm, f_bufs, q_ref, in_sems, out_sems):
    b = core * per + j
    _, C, H, W = f_bufs.shape
    nch = H // HC
    fP = jnp.float32(H * W)
    cur = jax.lax.rem(j, 3)
    nxt = jax.lax.rem(j + 1, 3)
    x_ref = f_bufs.at[cur]

    cps_in = _slab_copies(f_hbm, b, f_bufs, cur, in_sems)

    @pl.when(j == 0)
    def _():
        for cp in cps_in:
            cp.start()

    for cp in cps_in:
        cp.wait()

    def phase_a(i, acc):
        fc = x_ref[:, pl.ds(i * HC, HC), :]
        return acc + jnp.sum(fc, axis=1)

    acc = jax.lax.fori_loop(0, nch, phase_a, jnp.zeros((C, W), jnp.float32))
    a_rep = jnp.dot(acc, jnp.ones((W, W), jnp.float32),
                    precision=HIGH, preferred_element_type=jnp.float32) / fP
    na = jnp.maximum(
        jnp.sqrt(jnp.sum(a_rep * a_rep, axis=0, keepdims=True)), EPSV)
    an2 = a_rep / na

    kpat = jax.lax.broadcasted_iota(jnp.int32, (16 * HC, W), 0) // HC

    def onehots(q):
        hi_t = jnp.tile(q >> 4, (16, 1))
        lo_t = jnp.tile(q & 15, (16, 1))
        oh_hi = jnp.where(hi_t == kpat, 1.0, 0.0)
        oh_lo = jnp.where(lo_t == kpat, 1.0, 0.0)
        return oh_hi, oh_lo

    def phase_b(i, m2):
        ds = pl.ds(i * HC, HC)
        fc = x_ref[:, ds, :]
        dotc = jnp.sum(fc * an2[:, None, :], axis=0)
        ssq = jnp.sum(fc * fc, axis=0)
        npx = jnp.maximum(jnp.sqrt(ssq), EPSV)
        cos = dotc / npx
        qi = (cos * 255.0).astype(jnp.int32)
        q = (qi + 256) & 255
        q_ref[ds, :] = q.astype(jnp.float32)
        oh_hi, oh_lo = onehots(q)
        return m2 + jax.lax.dot_general(
            oh_hi, oh_lo, (((1,), (1,)), ((), ())),
            preferred_element_type=jnp.float32)

    m2 = jax.lax.fori_loop(0, nch, phase_b,
                           jnp.zeros((16 * HC, 16 * HC), jnp.float32))

    @pl.when(j >= 2)
    def _():
        for cp in _out_copies(f_bufs, nxt, out_hbm, b - 2, out_sems):
            cp.wait()

    @pl.when(j < per - 1)
    def _():
        for cp in _slab_copies(f_hbm, b + 1, f_bufs, nxt, in_sems):
            cp.start()

    n2 = 16 * HC
    i2r = jax.lax.broadcasted_iota(jnp.int32, (n2, n2), 0)
    i2c = jax.lax.broadcasted_iota(jnp.int32, (n2, n2), 1)
    dmask = jnp.where(jax.lax.rem(i2r, HC) == jax.lax.rem(i2c, HC), 1.0, 0.0)
    s16r = jax.lax.broadcasted_iota(jnp.int32, (16, n2), 0)
    s16c = jax.lax.broadcasted_iota(jnp.int32, (16, n2), 1)
    smat = jnp.where(s16r == s16c // HC, 1.0, 0.0)
    hist = jnp.dot(jnp.dot(smat, m2 * dmask, precision=HIGH,
                           preferred_element_type=jnp.float32),
                   smat.T, precision=HIGH,
                   preferred_element_type=jnp.float32)

    r16 = jax.lax.broadcasted_iota(jnp.int32, (16, 16), 0)
    c16 = jax.lax.broadcasted_iota(jnp.int32, (16, 16), 1)
    upper = jnp.where(r16 <= c16, 1.0, 0.0)
    lstrict = jnp.where(c16 < r16, 1.0, 0.0)
    cdf_lo = jnp.dot(hist, upper, precision=HIGH,
                     preferred_element_type=jnp.float32)
    rowsum = jnp.sum(hist, axis=1, keepdims=True)
    offs = jnp.dot(lstrict, rowsum, precision=HIGH,
                   preferred_element_type=jnp.float32)
    cdf = cdf_lo + offs
    masked = jnp.where(hist > 0.0, cdf, fP + 1.0)
    cmin = jnp.min(jnp.min(masked, axis=1, keepdims=True),
                   axis=0, keepdims=True)
    denom = jnp.maximum(fP - cmin, 1.0)
    lut = jnp.clip(jnp.round((cdf - cmin) * (255.0 / denom)), 0.0, 255.0)

    lut_d = jnp.dot(jnp.dot(smat.T, lut,
                            preferred_element_type=jnp.float32),
                    smat, preferred_element_type=jnp.float32) * dmask

    def phase_d(i, carry):
        ds = pl.ds(i * HC, HC)
        q = q_ref[ds, :].astype(jnp.int32)
        oh_hi, oh_lo = onehots(q)
        z = jnp.dot(lut_d, oh_lo, preferred_element_type=jnp.float32)
        prod = oh_hi * z
        equ = prod[0:HC, :]
        for k in range(1, 16):
            equ = equ + prod[k * HC:(k + 1) * HC, :]
        scale = equ * jnp.float32(1.0 / 255.0)
        x_ref[:, ds, :] = x_ref[:, ds, :] * scale[None, :, :]
        return carry

    jax.lax.fori_loop(0, nch, phase_d, 0)

    for cp in _out_copies(f_bufs, cur, out_hbm, b, out_sems):
        cp.start()

    @pl.when(j == per - 1)
    def _():
        for cp in _out_copies(f_bufs, cur, out_hbm, b, out_sems):
            cp.wait()

    if per >= 2:
        @pl.when(j == per - 1)
        def _():
            prv = jax.lax.rem(j - 1, 3)
            for cp in _out_copies(f_bufs, prv, out_hbm, b - 1, out_sems):
                cp.wait()


@functools.partial(jax.jit, static_argnames=("ncores",))
def _run(F_S, ncores):
    B, C, H, W = F_S.shape
    per = B // ncores
    mesh = pltpu.create_tensorcore_mesh("core", num_cores=ncores)

    def inner(refs):
        f_ref, o_ref = refs

        @pl.core_map(mesh)
        def _():
            core = jax.lax.axis_index("core")

            def scoped(f_bufs, q_ref, in_sems, out_sems):
                def body(j, carry):
                    _step(j, core, per, f_ref, o_ref, f_bufs, q_ref,
                          in_sems, out_sems)
                    return carry

                jax.lax.fori_loop(0, per, body, 0)

            pl.run_scoped(
                scoped,
                pltpu.VMEM((3, C, H, W), jnp.float32),
                pltpu.VMEM((H, W), jnp.float32),
                pltpu.SemaphoreType.DMA((3,)),
                pltpu.SemaphoreType.DMA((3,)),
            )

    _, out = pl.run_state(inner)(
        (F_S, pl.empty((B, C, H, W), jnp.float32)))
    return out


def kernel(F_S):
    dev = jax.devices()[0]
    ncores = getattr(dev, "num_cores", 1) or 1
    if F_S.shape[0] % ncores != 0:
        ncores = 1
    return _run(F_S, ncores)


# speedup vs baseline: 1.0674x; 1.0674x over previous
import functools

import jax
import jax.numpy as jnp
from jax.experimental import pallas as pl
from jax.experimental.pallas import tpu as pltpu

EPSV = 1e-12
HC = 16
HIGH = jax.lax.Precision.HIGHEST
NSPLIT = 4


def _slab_copies(hbm, b, f_bufs, slot, sems):
    C = f_bufs.shape[1]
    cs = C // NSPLIT
    return [
        pltpu.make_async_copy(hbm.at[b, pl.ds(s * cs, cs)],
                              f_bufs.at[slot, pl.ds(s * cs, cs)],
                              sems.at[slot])
        for s in range(NSPLIT)
    ]


def _out_copies(f_bufs, slot, hbm, b, sems):
    C = f_bufs.shape[1]
    cs = C // NSPLIT
    return [
        pltpu.make_async_copy(f_bufs.at[slot, pl.ds(s * cs, cs)],
                              hbm.at[b, pl.ds(s * cs, cs)],
                              sems.at[slot])
        for s in range(NSPLIT)
    ]


def _step(j, core, per, f_hbm, out_hbm, f_bufs, q_ref, in_sems, out_sems):
    b = core * per + j
    _, C, H, W = f_bufs.shape
    nch = H // HC
    fP = jnp.float32(H * W)
    cur = jax.lax.rem(j, 3)
    nxt = jax.lax.rem(j + 1, 3)
    x_ref = f_bufs.at[cur]

    cps_in = _slab_copies(f_hbm, b, f_bufs, cur, in_sems)

    @pl.when(j == 0)
    def _():
        for cp in cps_in:
            cp.start()

    for cp in cps_in:
        cp.wait()

    def phase_a(i, acc):
        fc = x_ref[:, pl.ds(i * HC, HC), :]
        return acc + jnp.sum(fc, axis=1)

    acc = jax.lax.fori_loop(0, nch, phase_a, jnp.zeros((C, W), jnp.float32))
    a_rep = jnp.dot(acc, jnp.ones((W, W), jnp.float32),
                    precision=HIGH, preferred_element_type=jnp.float32) / fP
    na = jnp.maximum(
        jnp.sqrt(jnp.sum(a_rep * a_rep, axis=0, keepdims=True)), EPSV)
    an2 = a_rep / na

    kpat = jax.lax.broadcasted_iota(jnp.int32, (16 * HC, W), 0) // HC

    def onehots(q):
        hi_t = jnp.tile(q >> 4, (16, 1))
        lo_t = jnp.tile(q & 15, (16, 1))
        oh_hi = jnp.where(hi_t == kpat, 1.0, 0.0)
        oh_lo = jnp.where(lo_t == kpat, 1.0, 0.0)
        return oh_hi, oh_lo

    def phase_b(i, m2):
        ds = pl.ds(i * HC, HC)
        fc = x_ref[:, ds, :]
        dotc = jnp.sum(fc * an2[:, None, :], axis=0)
        ssq = jnp.sum(fc * fc, axis=0)
        npx = jnp.maximum(jnp.sqrt(ssq), EPSV)
        cos = dotc / npx
        qi = (cos * 255.0).astype(jnp.int32)
        q = (qi + 256) & 255
        q_ref[ds, :] = q.astype(jnp.float32)
        oh_hi, oh_lo = onehots(q)
        return m2 + jax.lax.dot_general(
            oh_hi, oh_lo, (((1,), (1,)), ((), ())),
            preferred_element_type=jnp.float32)

    m2 = jax.lax.fori_loop(0, nch, phase_b,
                           jnp.zeros((16 * HC, 16 * HC), jnp.float32))

    @pl.when(j >= 2)
    def _():
        for cp in _out_copies(f_bufs, nxt, out_hbm, b - 2, out_sems):
            cp.wait()

    @pl.when(j < per - 1)
    def _():
        for cp in _slab_copies(f_hbm, b + 1, f_bufs, nxt, in_sems):
            cp.start()

    n2 = 16 * HC
    i2r = jax.lax.broadcasted_iota(jnp.int32, (n2, n2), 0)
    i2c = jax.lax.broadcasted_iota(jnp.int32, (n2, n2), 1)
    dmask = jnp.where(jax.lax.rem(i2r, HC) == jax.lax.rem(i2c, HC), 1.0, 0.0)
    s16r = jax.lax.broadcasted_iota(jnp.int32, (16, n2), 0)
    s16c = jax.lax.broadcasted_iota(jnp.int32, (16, n2), 1)
    smat = jnp.where(s16r == s16c // HC, 1.0, 0.0)
    hist = jnp.dot(jnp.dot(smat, m2 * dmask, precision=HIGH,
                           preferred_element_type=jnp.float32),
                   smat.T, precision=HIGH,
                   preferred_element_type=jnp.float32)

    r16 = jax.lax.broadcasted_iota(jnp.int32, (16, 16), 0)
    c16 = jax.lax.broadcasted_iota(jnp.int32, (16, 16), 1)
    upper = jnp.where(r16 <= c16, 1.0, 0.0)
    lstrict = jnp.where(c16 < r16, 1.0, 0.0)
    cdf_lo = jnp.dot(hist, upper, precision=HIGH,
                     preferred_element_type=jnp.float32)
    rowsum = jnp.sum(hist, axis=1, keepdims=True)
    offs = jnp.dot(lstrict, rowsum, precision=HIGH,
                   preferred_element_type=jnp.float32)
    cdf = cdf_lo + offs
    masked = jnp.where(hist > 0.0, cdf, fP + 1.0)
    cmin = jnp.min(jnp.min(masked, axis=1, keepdims=True),
                   axis=0, keepdims=True)
    denom = jnp.maximum(fP - cmin, 1.0)
    lut = jnp.clip(jnp.round((cdf - cmin) * (255.0 / denom)), 0.0, 255.0)

    lut1d = jnp.concatenate([lut[k:k + 1, :] for k in range(16)], axis=1)
    tbl_lo = jnp.broadcast_to(lut1d[:, 0:128], (HC, 128))
    tbl_hi = jnp.broadcast_to(lut1d[:, 128:256], (HC, 128))

    def phase_d(i, carry):
        ds = pl.ds(i * HC, HC)
        q = q_ref[ds, :].astype(jnp.int32)
        qm = q & 127
        g_lo = jnp.take_along_axis(tbl_lo, qm, axis=1)
        g_hi = jnp.take_along_axis(tbl_hi, qm, axis=1)
        equ = jnp.where(q < 128, g_lo, g_hi)
        scale = equ * jnp.float32(1.0 / 255.0)
        x_ref[:, ds, :] = x_ref[:, ds, :] * scale[None, :, :]
        return carry

    jax.lax.fori_loop(0, nch, phase_d, 0)

    for cp in _out_copies(f_bufs, cur, out_hbm, b, out_sems):
        cp.start()

    @pl.when(j == per - 1)
    def _():
        for cp in _out_copies(f_bufs, cur, out_hbm, b, out_sems):
            cp.wait()

    if per >= 2:
        @pl.when(j == per - 1)
        def _():
            prv = jax.lax.rem(j - 1, 3)
            for cp in _out_copies(f_bufs, prv, out_hbm, b - 1, out_sems):
                cp.wait()


@functools.partial(jax.jit, static_argnames=("ncores",))
def _run(F_S, ncores):
    B, C, H, W = F_S.shape
    per = B // ncores
    mesh = pltpu.create_tensorcore_mesh("core", num_cores=ncores)

    def inner(refs):
        f_ref, o_ref = refs

        @pl.core_map(mesh)
        def _():
            core = jax.lax.axis_index("core")

            def scoped(f_bufs, q_ref, in_sems, out_sems):
                def body(j, carry):
                    _step(j, core, per, f_ref, o_ref, f_bufs, q_ref,
                          in_sems, out_sems)
                    return carry

                jax.lax.fori_loop(0, per, body, 0)

            pl.run_scoped(
                scoped,
                pltpu.VMEM((3, C, H, W), jnp.float32),
                pltpu.VMEM((H, W), jnp.float32),
                pltpu.SemaphoreType.DMA((3,)),
                pltpu.SemaphoreType.DMA((3,)),
            )

    _, out = pl.run_state(inner)(
        (F_S, pl.empty((B, C, H, W), jnp.float32)))
    return out


def kernel(F_S):
    dev = jax.devices()[0]
    ncores = getattr(dev, "num_cores", 1) or 1
    if F_S.shape[0] % ncores != 0:
        ncores = 1
    return _run(F_S, ncores)


# speedup vs baseline: 1.0895x; 1.0207x over previous
import functools

import jax
import jax.numpy as jnp
from jax.experimental import pallas as pl
from jax.experimental.pallas import tpu as pltpu

EPSV = 1e-12
HC = 16
HIGH = jax.lax.Precision.HIGHEST
NSPLIT = 4


def _slab_copies(hbm, b, f_bufs, slot, sems):
    C = f_bufs.shape[1]
    cs = C // NSPLIT
    return [
        pltpu.make_async_copy(hbm.at[b, pl.ds(s * cs, cs)],
                              f_bufs.at[slot, pl.ds(s * cs, cs)],
                              sems.at[slot])
        for s in range(NSPLIT)
    ]


def _out_copies(f_bufs, slot, hbm, b, sems):
    C = f_bufs.shape[1]
    cs = C // NSPLIT
    return [
        pltpu.make_async_copy(f_bufs.at[slot, pl.ds(s * cs, cs)],
                              hbm.at[b, pl.ds(s * cs, cs)],
                              sems.at[slot])
        for s in range(NSPLIT)
    ]


def _step(j, core, per, f_hbm, out_hbm, f_bufs, q_ref, in_sems, out_sems):
    b = core * per + j
    _, C, H, W = f_bufs.shape
    nch = H // HC
    fP = jnp.float32(H * W)
    cur = jax.lax.rem(j, 3)
    nxt = jax.lax.rem(j + 1, 3)
    x_ref = f_bufs.at[cur]

    cps_in = _slab_copies(f_hbm, b, f_bufs, cur, in_sems)

    @pl.when(j == 0)
    def _():
        for cp in cps_in:
            cp.start()

    for cp in cps_in:
        cp.wait()

    def phase_a(i, acc):
        fc = x_ref[:, pl.ds(i * HC, HC), :]
        return acc + jnp.sum(fc, axis=1)

    acc = jax.lax.fori_loop(0, nch, phase_a, jnp.zeros((C, W), jnp.float32))
    a_rep = jnp.dot(acc, jnp.ones((W, W), jnp.float32),
                    precision=HIGH, preferred_element_type=jnp.float32) / fP
    na = jnp.maximum(
        jnp.sqrt(jnp.sum(a_rep * a_rep, axis=0, keepdims=True)), EPSV)
    an3 = jnp.broadcast_to((a_rep / na)[:, None, :], (C, HC, W))

    kpat = jax.lax.broadcasted_iota(jnp.int32, (16 * HC, W), 0) // HC

    def onehots(q):
        hi_t = jnp.tile(q >> 4, (16, 1))
        lo_t = jnp.tile(q & 15, (16, 1))
        oh_hi = jnp.where(hi_t == kpat, 1.0, 0.0)
        oh_lo = jnp.where(lo_t == kpat, 1.0, 0.0)
        return oh_hi, oh_lo

    def phase_b(i, m2):
        ds = pl.ds(i * HC, HC)
        fc = x_ref[:, ds, :]
        dotc = jnp.sum(fc * an3, axis=0)
        ssq = jnp.sum(fc * fc, axis=0)
        npx = jnp.maximum(jnp.sqrt(ssq), EPSV)
        cos = dotc / npx
        qi = (cos * 255.0).astype(jnp.int32)
        q = (qi + 256) & 255
        q_ref[ds, :] = q.astype(jnp.float32)
        oh_hi, oh_lo = onehots(q)
        return m2 + jax.lax.dot_general(
            oh_hi, oh_lo, (((1,), (1,)), ((), ())),
            preferred_element_type=jnp.float32)

    m2 = jax.lax.fori_loop(0, nch, phase_b,
                           jnp.zeros((16 * HC, 16 * HC), jnp.float32))

    @pl.when(j >= 2)
    def _():
        for cp in _out_copies(f_bufs, nxt, out_hbm, b - 2, out_sems):
            cp.wait()

    @pl.when(j < per - 1)
    def _():
        for cp in _slab_copies(f_hbm, b + 1, f_bufs, nxt, in_sems):
            cp.start()

    n2 = 16 * HC
    i2r = jax.lax.broadcasted_iota(jnp.int32, (n2, n2), 0)
    i2c = jax.lax.broadcasted_iota(jnp.int32, (n2, n2), 1)
    dmask = jnp.where(jax.lax.rem(i2r, HC) == jax.lax.rem(i2c, HC), 1.0, 0.0)
    s16r = jax.lax.broadcasted_iota(jnp.int32, (16, n2), 0)
    s16c = jax.lax.broadcasted_iota(jnp.int32, (16, n2), 1)
    smat = jnp.where(s16r == s16c // HC, 1.0, 0.0)
    hist = jnp.dot(jnp.dot(smat, m2 * dmask, precision=HIGH,
                           preferred_element_type=jnp.float32),
                   smat.T, precision=HIGH,
                   preferred_element_type=jnp.float32)

    r16 = jax.lax.broadcasted_iota(jnp.int32, (16, 16), 0)
    c16 = jax.lax.broadcasted_iota(jnp.int32, (16, 16), 1)
    upper = jnp.where(r16 <= c16, 1.0, 0.0)
    lstrict = jnp.where(c16 < r16, 1.0, 0.0)
    cdf_lo = jnp.dot(hist, upper, precision=HIGH,
                     preferred_element_type=jnp.float32)
    rowsum = jnp.sum(hist, axis=1, keepdims=True)
    offs = jnp.dot(lstrict, rowsum, precision=HIGH,
                   preferred_element_type=jnp.float32)
    cdf = cdf_lo + offs
    masked = jnp.where(hist > 0.0, cdf, fP + 1.0)
    cmin = jnp.min(jnp.min(masked, axis=1, keepdims=True),
                   axis=0, keepdims=True)
    denom = jnp.maximum(fP - cmin, 1.0)
    lut = jnp.clip(jnp.round((cdf - cmin) * (255.0 / denom)), 0.0, 255.0)

    lut1d = jnp.concatenate([lut[k:k + 1, :] for k in range(16)], axis=1)
    tbl_lo = jnp.broadcast_to(lut1d[:, 0:128], (HC, 128))
    tbl_hi = jnp.broadcast_to(lut1d[:, 128:256], (HC, 128))

    def phase_d(i, carry):
        ds = pl.ds(i * HC, HC)
        q = q_ref[ds, :].astype(jnp.int32)
        qm = q & 127
        g_lo = jnp.take_along_axis(tbl_lo, qm, axis=1)
        g_hi = jnp.take_along_axis(tbl_hi, qm, axis=1)
        equ = jnp.where(q < 128, g_lo, g_hi)
        scale = equ * jnp.float32(1.0 / 255.0)
        x_ref[:, ds, :] = x_ref[:, ds, :] * scale[None, :, :]
        return carry

    jax.lax.fori_loop(0, nch, phase_d, 0)

    for cp in _out_copies(f_bufs, cur, out_hbm, b, out_sems):
        cp.start()

    @pl.when(j == per - 1)
    def _():
        for cp in _out_copies(f_bufs, cur, out_hbm, b, out_sems):
            cp.wait()

    if per >= 2:
        @pl.when(j == per - 1)
        def _():
            prv = jax.lax.rem(j - 1, 3)
            for cp in _out_copies(f_bufs, prv, out_hbm, b - 1, out_sems):
                cp.wait()


@functools.partial(jax.jit, static_argnames=("ncores",))
def _run(F_S, ncores):
    B, C, H, W = F_S.shape
    per = B // ncores
    mesh = pltpu.create_tensorcore_mesh("core", num_cores=ncores)

    def inner(refs):
        f_ref, o_ref = refs

        @pl.core_map(mesh)
        def _():
            core = jax.lax.axis_index("core")

            def scoped(f_bufs, q_ref, in_sems, out_sems):
                def body(j, carry):
                    _step(j, core, per, f_ref, o_ref, f_bufs, q_ref,
                          in_sems, out_sems)
                    return carry

                jax.lax.fori_loop(0, per, body, 0)

            pl.run_scoped(
                scoped,
                pltpu.VMEM((3, C, H, W), jnp.float32),
                pltpu.VMEM((H, W), jnp.float32),
                pltpu.SemaphoreType.DMA((3,)),
                pltpu.SemaphoreType.DMA((3,)),
            )

    _, out = pl.run_state(inner)(
        (F_S, pl.empty((B, C, H, W), jnp.float32)))
    return out


def kernel(F_S):
    dev = jax.devices()[0]
    ncores = getattr(dev, "num_cores", 1) or 1
    if F_S.shape[0] % ncores != 0:
        ncores = 1
    return _run(F_S, ncores)
